# baseline (speedup 1.0000x reference)
import functools

import jax
import jax.numpy as jnp
from jax import lax
from jax.experimental import pallas as pl
from jax.experimental.pallas import tpu as pltpu
from jax.experimental.pallas import tpu_sc as plsc

NUM_EXPERTS = 8
TOP_K = 2
EMB = 1024
MLP = 4096

TM = 512
P_MAX = ((2048 * TOP_K + NUM_EXPERTS * (TM - 1)) + TM - 1) // TM * TM
U_MAX = P_MAX // TM
TN_UP = 2048
TN_DN = 1024
TB = 512


def _route_body(logits_ref, pos0_ref, pos1_ref, wa_ref, wb_ref,
                be_ref, vcnt_ref, nb_ref, carry_ref, r0s_ref, r1s_ref,
                poff_ref, pincl_ref):
    p = pl.program_id(0)
    g = pl.program_id(1)
    first = jnp.logical_and(p == 0, g == 0)

    @pl.when(first)
    def _():
        carry_ref[...] = jnp.zeros_like(carry_ref)

    logits = logits_ref[...]
    idx = lax.broadcasted_iota(jnp.int32, (TB, NUM_EXPERTS), 1)
    m1 = jnp.max(logits, axis=1, keepdims=True)
    a1 = jnp.min(jnp.where(logits == m1, idx, NUM_EXPERTS), axis=1)
    not1 = idx != a1[:, None]
    m2 = jnp.max(jnp.where(not1, logits, -jnp.inf), axis=1, keepdims=True)
    a2 = jnp.min(jnp.where((logits == m2) & not1, idx, NUM_EXPERTS), axis=1)
    wa = jax.nn.sigmoid(m1 - m2)[:, 0]
    idx16 = lax.broadcasted_iota(jnp.int32, (TB, 16), 1)
    oh1 = (idx16 == a1[:, None])
    oh2 = (idx16 == a2[:, None])

    wa_ref[...] = wa
    wb_ref[...] = 1.0 - wa

    @pl.when(p == 0)
    def _():
        oh = (oh1 | oh2).astype(jnp.float32)
        lt = (lax.broadcasted_iota(jnp.int32, (TB, TB), 0)
              > lax.broadcasted_iota(jnp.int32, (TB, TB), 1)
              ).astype(jnp.bfloat16)
        c_excl = carry_ref[...] + jnp.dot(lt, oh.astype(jnp.bfloat16),
                                          preferred_element_type=jnp.float32)
        r0s_ref[pl.ds(g * TB, TB)] = jnp.sum(jnp.where(oh1, c_excl, 0.0),
                                             axis=1)
        r1s_ref[pl.ds(g * TB, TB)] = jnp.sum(jnp.where(oh2, c_excl, 0.0),
                                             axis=1)
        carry_ref[...] += jnp.sum(oh, axis=0, keepdims=True)

        @pl.when(g == pl.num_programs(1) - 1)
        def _():
            cnt = carry_ref[...]
            padded = jnp.floor((cnt + (TM - 1)) * (1.0 / TM)) * TM
            le = (lax.broadcasted_iota(jnp.int32, (16, 16), 0)
                  <= lax.broadcasted_iota(jnp.int32, (16, 16), 1)
                  ).astype(jnp.bfloat16)
            pincl = jnp.dot(padded.astype(jnp.bfloat16), le,
                            preferred_element_type=jnp.float32)
            pincl_ref[...] = pincl
            poff_ref[...] = pincl - padded

    @pl.when(p == 1)
    def _():
        poff = poff_ref[...]
        pofa1 = jnp.sum(jnp.where(oh1, poff, 0.0), axis=1)
        pofa2 = jnp.sum(jnp.where(oh2, poff, 0.0), axis=1)
        pos0_ref[...] = (pofa1 + r0s_ref[pl.ds(g * TB, TB)]).astype(jnp.int32)
        pos1_ref[...] = (pofa2 + r1s_ref[pl.ds(g * TB, TB)]).astype(jnp.int32)
        bu = (lax.broadcasted_iota(jnp.int32, (32, 16), 0) * TM
              ).astype(jnp.float32)
        nfull = jnp.sum((bu >= pincl_ref[...]).astype(jnp.int32), axis=1)
        be = jnp.minimum(nfull, NUM_EXPERTS - 1)
        be_ref[...] = be
        ohb = (lax.broadcasted_iota(jnp.int32, (32, 16), 1) == be[:, None])
        poff_sel = jnp.sum(jnp.where(ohb, poff, 0.0), axis=1)
        cnt_sel = jnp.sum(jnp.where(ohb, carry_ref[...], 0.0), axis=1)
        vcnt = jnp.clip(poff_sel + cnt_sel - bu[:, 0], 0.0, float(TM))
        vcnt_ref[...] = vcnt.astype(jnp.int32)
        nb_ref[...] = (pincl_ref[0, 15:16] * (1.0 / TM)).astype(jnp.int32)


def _route(logits):
    T = logits.shape[0]
    n = T // TB
    vec = lambda d: jax.ShapeDtypeStruct((T,), d)
    return pl.pallas_call(
        _route_body,
        grid=(2, n),
        in_specs=[
            pl.BlockSpec((TB, NUM_EXPERTS), lambda p, g: (g, 0)),
        ],
        out_specs=[pl.BlockSpec((TB,), lambda p, g: (g,))] * 4
        + [pl.BlockSpec((32,), lambda p, g: (0,))] * 2
        + [pl.BlockSpec((1,), lambda p, g: (0,))],
        out_shape=[vec(jnp.int32), vec(jnp.int32), vec(jnp.float32),
                   vec(jnp.float32),
                   jax.ShapeDtypeStruct((32,), jnp.int32),
                   jax.ShapeDtypeStruct((32,), jnp.int32),
                   jax.ShapeDtypeStruct((1,), jnp.int32)],
        scratch_shapes=[pltpu.VMEM((1, 16), jnp.float32),
                        pltpu.VMEM((T,), jnp.float32),
                        pltpu.VMEM((T,), jnp.float32),
                        pltpu.VMEM((1, 16), jnp.float32),
                        pltpu.VMEM((1, 16), jnp.float32)],
    )(logits)


def _up_body(meta_ref, x_ref, w0_ref, w1_ref, out_ref):
    x = x_ref[...].astype(jnp.bfloat16)
    a0 = jnp.dot(x, w0_ref[0].astype(jnp.bfloat16),
                 preferred_element_type=jnp.float32)
    a1 = jnp.dot(x, w1_ref[0].astype(jnp.bfloat16),
                 preferred_element_type=jnp.float32)
    out_ref[...] = ((a0 * jax.nn.sigmoid(a0)) * a1).astype(jnp.bfloat16)


def _dn_body(meta_ref, vcnt_ref, x_ref, wo_ref, ws_ref, out_ref):
    u = pl.program_id(1)
    acc = jnp.dot(x_ref[...], wo_ref[0].astype(jnp.bfloat16),
                  preferred_element_type=jnp.float32)
    rows = lax.broadcasted_iota(jnp.int32, (TM, 1), 0)
    valid = rows < vcnt_ref[u]
    out_ref[...] = jnp.where(valid, acc * ws_ref[...][:, None], 0.0)


def _up_gmm(meta, xs, w0, w1, nblocks):
    grid = (MLP // TN_UP, nblocks)
    return pl.pallas_call(
        _up_body,
        grid_spec=pltpu.PrefetchScalarGridSpec(
            num_scalar_prefetch=1,
            grid=grid,
            in_specs=[
                pl.BlockSpec((TM, EMB), lambda n, u, m: (u, 0)),
                pl.BlockSpec((1, EMB, TN_UP), lambda n, u, m: (m[u], 0, n)),
                pl.BlockSpec((1, EMB, TN_UP), lambda n, u, m: (m[u], 0, n)),
            ],
            out_specs=pl.BlockSpec((TM, TN_UP), lambda n, u, m: (u, n)),
        ),
        out_shape=jax.ShapeDtypeStruct((P_MAX, MLP), jnp.bfloat16),
    )(meta, xs, w0, w1)


def _dn_gmm(meta, vcnt, inter, wo, ws, nblocks):
    grid = (EMB // TN_DN, nblocks)
    return pl.pallas_call(
        _dn_body,
        grid_spec=pltpu.PrefetchScalarGridSpec(
            num_scalar_prefetch=2,
            grid=grid,
            in_specs=[
                pl.BlockSpec((TM, MLP), lambda n, u, m, v: (u, 0)),
                pl.BlockSpec((1, MLP, TN_DN), lambda n, u, m, v: (m[u], 0, n)),
                pl.BlockSpec((TM,), lambda n, u, m, v: (u,)),
            ],
            out_specs=pl.BlockSpec((TM, TN_DN), lambda n, u, m, v: (u, n)),
        ),
        out_shape=jax.ShapeDtypeStruct((P_MAX, EMB), jnp.float32),
    )(meta, vcnt, inter, wo, ws)


_NC, _NS = 2, 16
_NW = _NC * _NS
_CH = 32
_RW = P_MAX // _NW
_GCH = _RW // 2


def _dispatch_sc(x2, pos0, pos1, wa, wb):
    T = pos0.shape[0]
    tw = T // _NW

    mesh = plsc.VectorSubcoreMesh(core_axis_name="c", subcore_axis_name="s")

    @functools.partial(
        pl.kernel, mesh=mesh,
        out_type=[
            jax.ShapeDtypeStruct((P_MAX, EMB), jnp.float32),
            jax.ShapeDtypeStruct((P_MAX,), jnp.float32),
        ],
        scratch_types=[
            pltpu.VMEM((tw, EMB), jnp.float32),
            pltpu.VMEM((tw,), jnp.int32),
            pltpu.VMEM((tw,), jnp.int32),
            pltpu.VMEM((tw,), jnp.float32),
            pltpu.VMEM((tw,), jnp.float32),
            pltpu.SemaphoreType.DMA,
            pltpu.SemaphoreType.DMA,
            pltpu.SemaphoreType.DMA,
            pltpu.SemaphoreType.DMA,
        ],
    )
    def body(x2_h, p0_h, p1_h, wa_h, wb_h, xs_h, ws_h,
             rows_v, i0_v, i1_v, wav_v, wbv_v, sem0, sem1, sem2, sem3):
        c = lax.axis_index("c")
        s = lax.axis_index("s")
        wid = s * _NC + c
        base = wid * tw
        ld0 = pltpu.async_copy(x2_h.at[pl.ds(base, tw)], rows_v, sem0)
        ld1 = pltpu.async_copy(p0_h.at[pl.ds(base, tw)], i0_v, sem1)
        ld2 = pltpu.async_copy(p1_h.at[pl.ds(base, tw)], i1_v, sem2)
        ld3 = pltpu.async_copy(wa_h.at[pl.ds(base, tw)], wav_v, sem3)
        ld0.wait()
        ld1.wait()
        ld2.wait()
        ld3.wait()
        cp0 = pltpu.async_copy(rows_v, xs_h.at[i0_v], sem0)
        cp1 = pltpu.async_copy(rows_v, xs_h.at[i1_v], sem1)
        cp2 = pltpu.async_copy(wav_v, ws_h.at[i0_v], sem2)
        ld4 = pltpu.async_copy(wb_h.at[pl.ds(base, tw)], wbv_v, sem3)
        ld4.wait()
        cp3 = pltpu.async_copy(wbv_v, ws_h.at[i1_v], sem3)
        cp0.wait()
        cp1.wait()
        cp2.wait()
        cp3.wait()

    return body(x2, pos0, pos1, wa, wb)


def _combine_sc(y, pos0, pos1):
    T = pos0.shape[0]
    per_w = T // _NW
    n_ch = per_w // _CH
    mesh = plsc.VectorSubcoreMesh(core_axis_name="c", subcore_axis_name="s")

    @functools.partial(
        pl.kernel, mesh=mesh,
        out_type=jax.ShapeDtypeStruct((T, EMB), jnp.float32),
        scratch_types=[
            pltpu.VMEM((_CH,), jnp.int32),
            pltpu.VMEM((_CH,), jnp.int32),
            pltpu.VMEM((_CH, EMB), jnp.float32),
            pltpu.VMEM((_CH, EMB), jnp.float32),
            pltpu.SemaphoreType.DMA,
            pltpu.SemaphoreType.DMA,
        ],
    )
    def body(y_hbm, p0_hbm, p1_hbm, out_hbm,
             idx0_v, idx1_v, rows0_v, rows1_v, sem0, sem1):
        wid = lax.axis_index("s") * _NC + lax.axis_index("c")
        for ch in range(n_ch):
            base = wid * per_w + ch * _CH
            pltpu.sync_copy(p0_hbm.at[pl.ds(base, _CH)], idx0_v)
            pltpu.sync_copy(p1_hbm.at[pl.ds(base, _CH)], idx1_v)
            cp0 = pltpu.async_copy(y_hbm.at[idx0_v], rows0_v, sem0)
            cp1 = pltpu.async_copy(y_hbm.at[idx1_v], rows1_v, sem1)
            cp0.wait()
            cp1.wait()

            def row_add(r, _):
                for k in range(EMB // 16):
                    sl = pl.ds(k * 16, 16)
                    rows0_v[r, sl] = rows0_v[r, sl] + rows1_v[r, sl]
                return 0

            lax.fori_loop(0, _CH, row_add, 0)
            pltpu.sync_copy(rows0_v, out_hbm.at[pl.ds(base, _CH)])

    return body(y, pos0, pos1)


def kernel(inputs, gate_kernel, w0_kernel, w1_kernel, wo_kernel):
    inputs = inputs.astype(jnp.float32)
    x2 = inputs.reshape(-1, EMB)
    T = x2.shape[0]

    logits = jnp.einsum('bsd,de->bse', inputs, gate_kernel).reshape(T, NUM_EXPERTS)
    pos0, pos1, wa, wb, block_expert, vcnt, nb = _route(logits)
    nblocks = nb[0]

    xs, ws = _dispatch_sc(x2, pos0, pos1, wa, wb)

    inter = _up_gmm(block_expert, xs, w0_kernel, w1_kernel, nblocks)
    y = _dn_gmm(block_expert, vcnt, inter, wo_kernel, ws, nblocks)

    out = _combine_sc(y, pos0, pos1)
    return out.reshape(inputs.shape)


# speedup vs baseline: 1.0144x; 1.0144x over previous
import functools

import jax
import jax.numpy as jnp
from jax import lax
from jax.experimental import pallas as pl
from jax.experimental.pallas import tpu as pltpu
from jax.experimental.pallas import tpu_sc as plsc

NUM_EXPERTS = 8
TOP_K = 2
EMB = 1024
MLP = 4096

TM = 512
P_MAX = ((2048 * TOP_K + NUM_EXPERTS * (TM - 1)) + TM - 1) // TM * TM
U_MAX = P_MAX // TM
TN_UP = 2048
TN_DN = 1024
TB = 512


def _route_body(logits_ref, pos0_ref, pos1_ref, wa_ref, wb_ref,
                be_ref, vcnt_ref, nb_ref, carry_ref, r0s_ref, r1s_ref,
                poff_ref, pincl_ref):
    p = pl.program_id(0)
    g = pl.program_id(1)
    first = jnp.logical_and(p == 0, g == 0)

    @pl.when(first)
    def _():
        carry_ref[...] = jnp.zeros_like(carry_ref)

    logits = logits_ref[...]
    idx = lax.broadcasted_iota(jnp.int32, (TB, NUM_EXPERTS), 1)
    m1 = jnp.max(logits, axis=1, keepdims=True)
    a1 = jnp.min(jnp.where(logits == m1, idx, NUM_EXPERTS), axis=1)
    not1 = idx != a1[:, None]
    m2 = jnp.max(jnp.where(not1, logits, -jnp.inf), axis=1, keepdims=True)
    a2 = jnp.min(jnp.where((logits == m2) & not1, idx, NUM_EXPERTS), axis=1)
    wa = jax.nn.sigmoid(m1 - m2)[:, 0]
    idx16 = lax.broadcasted_iota(jnp.int32, (TB, 16), 1)
    oh1 = (idx16 == a1[:, None])
    oh2 = (idx16 == a2[:, None])

    wa_ref[...] = wa
    wb_ref[...] = 1.0 - wa

    @pl.when(p == 0)
    def _():
        oh = (oh1 | oh2).astype(jnp.float32)
        lt = (lax.broadcasted_iota(jnp.int32, (TB, TB), 0)
              > lax.broadcasted_iota(jnp.int32, (TB, TB), 1)
              ).astype(jnp.bfloat16)
        c_excl = carry_ref[...] + jnp.dot(lt, oh.astype(jnp.bfloat16),
                                          preferred_element_type=jnp.float32)
        r0s_ref[pl.ds(g * TB, TB)] = jnp.sum(jnp.where(oh1, c_excl, 0.0),
                                             axis=1)
        r1s_ref[pl.ds(g * TB, TB)] = jnp.sum(jnp.where(oh2, c_excl, 0.0),
                                             axis=1)
        carry_ref[...] += jnp.sum(oh, axis=0, keepdims=True)

        @pl.when(g == pl.num_programs(1) - 1)
        def _():
            cnt = carry_ref[...]
            padded = jnp.floor((cnt + (TM - 1)) * (1.0 / TM)) * TM
            le = (lax.broadcasted_iota(jnp.int32, (16, 16), 0)
                  <= lax.broadcasted_iota(jnp.int32, (16, 16), 1)
                  ).astype(jnp.bfloat16)
            pincl = jnp.dot(padded.astype(jnp.bfloat16), le,
                            preferred_element_type=jnp.float32)
            pincl_ref[...] = pincl
            poff_ref[...] = pincl - padded

    @pl.when(p == 1)
    def _():
        poff = poff_ref[...]
        pofa1 = jnp.sum(jnp.where(oh1, poff, 0.0), axis=1)
        pofa2 = jnp.sum(jnp.where(oh2, poff, 0.0), axis=1)
        pos0_ref[...] = (pofa1 + r0s_ref[pl.ds(g * TB, TB)]).astype(jnp.int32)
        pos1_ref[...] = (pofa2 + r1s_ref[pl.ds(g * TB, TB)]).astype(jnp.int32)
        bu = (lax.broadcasted_iota(jnp.int32, (32, 16), 0) * TM
              ).astype(jnp.float32)
        nfull = jnp.sum((bu >= pincl_ref[...]).astype(jnp.int32), axis=1)
        be = jnp.minimum(nfull, NUM_EXPERTS - 1)
        be_ref[...] = be
        ohb = (lax.broadcasted_iota(jnp.int32, (32, 16), 1) == be[:, None])
        poff_sel = jnp.sum(jnp.where(ohb, poff, 0.0), axis=1)
        cnt_sel = jnp.sum(jnp.where(ohb, carry_ref[...], 0.0), axis=1)
        vcnt = jnp.clip(poff_sel + cnt_sel - bu[:, 0], 0.0, float(TM))
        vcnt_ref[...] = vcnt.astype(jnp.int32)
        nb_ref[...] = (pincl_ref[0, 15:16] * (1.0 / TM)).astype(jnp.int32)


def _route(logits):
    T = logits.shape[0]
    n = T // TB
    vec = lambda d: jax.ShapeDtypeStruct((T,), d)
    return pl.pallas_call(
        _route_body,
        grid=(2, n),
        in_specs=[
            pl.BlockSpec((TB, NUM_EXPERTS), lambda p, g: (g, 0)),
        ],
        out_specs=[pl.BlockSpec((TB,), lambda p, g: (g,))] * 4
        + [pl.BlockSpec((32,), lambda p, g: (0,))] * 2
        + [pl.BlockSpec((1,), lambda p, g: (0,))],
        out_shape=[vec(jnp.int32), vec(jnp.int32), vec(jnp.float32),
                   vec(jnp.float32),
                   jax.ShapeDtypeStruct((32,), jnp.int32),
                   jax.ShapeDtypeStruct((32,), jnp.int32),
                   jax.ShapeDtypeStruct((1,), jnp.int32)],
        scratch_shapes=[pltpu.VMEM((1, 16), jnp.float32),
                        pltpu.VMEM((T,), jnp.float32),
                        pltpu.VMEM((T,), jnp.float32),
                        pltpu.VMEM((1, 16), jnp.float32),
                        pltpu.VMEM((1, 16), jnp.float32)],
    )(logits)


def _up_body(meta_ref, x_ref, w0_ref, w1_ref, out_ref):
    x = x_ref[...].astype(jnp.bfloat16)
    a0 = jnp.dot(x, w0_ref[0].astype(jnp.bfloat16),
                 preferred_element_type=jnp.float32)
    a1 = jnp.dot(x, w1_ref[0].astype(jnp.bfloat16),
                 preferred_element_type=jnp.float32)
    out_ref[...] = ((a0 * jax.nn.sigmoid(a0)) * a1).astype(jnp.bfloat16)


def _dn_body(meta_ref, vcnt_ref, x_ref, wo_ref, ws_ref, out_ref):
    u = pl.program_id(1)
    acc = jnp.dot(x_ref[...], wo_ref[0].astype(jnp.bfloat16),
                  preferred_element_type=jnp.float32)
    rows = lax.broadcasted_iota(jnp.int32, (TM, 1), 0)
    valid = rows < vcnt_ref[u]
    out_ref[...] = jnp.where(valid, acc * ws_ref[...][:, None], 0.0)


def _up_gmm(meta, xs, w0, w1, nblocks):
    grid = (MLP // TN_UP, nblocks)
    return pl.pallas_call(
        _up_body,
        grid_spec=pltpu.PrefetchScalarGridSpec(
            num_scalar_prefetch=1,
            grid=grid,
            in_specs=[
                pl.BlockSpec((TM, EMB), lambda n, u, m: (u, 0)),
                pl.BlockSpec((1, EMB, TN_UP), lambda n, u, m: (m[u], 0, n)),
                pl.BlockSpec((1, EMB, TN_UP), lambda n, u, m: (m[u], 0, n)),
            ],
            out_specs=pl.BlockSpec((TM, TN_UP), lambda n, u, m: (u, n)),
        ),
        out_shape=jax.ShapeDtypeStruct((P_MAX, MLP), jnp.bfloat16),
    )(meta, xs, w0, w1)


def _dn_gmm(meta, vcnt, inter, wo, ws, nblocks):
    grid = (EMB // TN_DN, nblocks)
    return pl.pallas_call(
        _dn_body,
        grid_spec=pltpu.PrefetchScalarGridSpec(
            num_scalar_prefetch=2,
            grid=grid,
            in_specs=[
                pl.BlockSpec((TM, MLP), lambda n, u, m, v: (u, 0)),
                pl.BlockSpec((1, MLP, TN_DN), lambda n, u, m, v: (m[u], 0, n)),
                pl.BlockSpec((TM,), lambda n, u, m, v: (u,)),
            ],
            out_specs=pl.BlockSpec((TM, TN_DN), lambda n, u, m, v: (u, n)),
        ),
        out_shape=jax.ShapeDtypeStruct((P_MAX, EMB), jnp.float32),
    )(meta, vcnt, inter, wo, ws)


_NC, _NS = 2, 16
_NW = _NC * _NS
_CH = 32
_RW = P_MAX // _NW
_GCH = _RW // 2


def _dispatch_sc(x2, pos0, pos1, wa, wb):
    T = pos0.shape[0]
    tw = T // _NW

    mesh = plsc.VectorSubcoreMesh(core_axis_name="c", subcore_axis_name="s")

    @functools.partial(
        pl.kernel, mesh=mesh,
        out_type=[
            jax.ShapeDtypeStruct((P_MAX, EMB), jnp.float32),
            jax.ShapeDtypeStruct((P_MAX,), jnp.float32),
        ],
        scratch_types=[
            pltpu.VMEM((tw, EMB), jnp.float32),
            pltpu.VMEM((tw,), jnp.int32),
            pltpu.VMEM((tw,), jnp.int32),
            pltpu.VMEM((tw,), jnp.float32),
            pltpu.VMEM((tw,), jnp.float32),
            pltpu.SemaphoreType.DMA,
            pltpu.SemaphoreType.DMA,
            pltpu.SemaphoreType.DMA,
            pltpu.SemaphoreType.DMA,
        ],
    )
    def body(x2_h, p0_h, p1_h, wa_h, wb_h, xs_h, ws_h,
             rows_v, i0_v, i1_v, wav_v, wbv_v, sem0, sem1, sem2, sem3):
        c = lax.axis_index("c")
        s = lax.axis_index("s")
        wid = s * _NC + c
        base = wid * tw
        pltpu.sync_copy(x2_h.at[pl.ds(base, tw)], rows_v)
        pltpu.sync_copy(p0_h.at[pl.ds(base, tw)], i0_v)
        pltpu.sync_copy(p1_h.at[pl.ds(base, tw)], i1_v)
        pltpu.sync_copy(wa_h.at[pl.ds(base, tw)], wav_v)
        pltpu.sync_copy(wb_h.at[pl.ds(base, tw)], wbv_v)
        cp0 = pltpu.async_copy(rows_v, xs_h.at[i0_v], sem0)
        cp1 = pltpu.async_copy(rows_v, xs_h.at[i1_v], sem1)
        cp2 = pltpu.async_copy(wav_v, ws_h.at[i0_v], sem2)
        cp3 = pltpu.async_copy(wbv_v, ws_h.at[i1_v], sem3)
        cp0.wait()
        cp1.wait()
        cp2.wait()
        cp3.wait()

    return body(x2, pos0, pos1, wa, wb)


def _combine_sc(y, pos0, pos1):
    T = pos0.shape[0]
    per_w = T // _NW
    n_ch = per_w // _CH
    mesh = plsc.VectorSubcoreMesh(core_axis_name="c", subcore_axis_name="s")

    @functools.partial(
        pl.kernel, mesh=mesh,
        out_type=jax.ShapeDtypeStruct((T, EMB), jnp.float32),
        scratch_types=[
            pltpu.VMEM((_CH,), jnp.int32),
            pltpu.VMEM((_CH,), jnp.int32),
            pltpu.VMEM((_CH, EMB), jnp.float32),
            pltpu.VMEM((_CH, EMB), jnp.float32),
            pltpu.SemaphoreType.DMA,
            pltpu.SemaphoreType.DMA,
        ],
    )
    def body(y_hbm, p0_hbm, p1_hbm, out_hbm,
             idx0_v, idx1_v, rows0_v, rows1_v, sem0, sem1):
        wid = lax.axis_index("s") * _NC + lax.axis_index("c")
        for ch in range(n_ch):
            base = wid * per_w + ch * _CH
            pltpu.sync_copy(p0_hbm.at[pl.ds(base, _CH)], idx0_v)
            pltpu.sync_copy(p1_hbm.at[pl.ds(base, _CH)], idx1_v)
            cp0 = pltpu.async_copy(y_hbm.at[idx0_v], rows0_v, sem0)
            cp1 = pltpu.async_copy(y_hbm.at[idx1_v], rows1_v, sem1)
            cp0.wait()
            cp1.wait()

            def row_add(r, _):
                for k in range(EMB // 16):
                    sl = pl.ds(k * 16, 16)
                    rows0_v[r, sl] = rows0_v[r, sl] + rows1_v[r, sl]
                return 0

            lax.fori_loop(0, _CH, row_add, 0)
            pltpu.sync_copy(rows0_v, out_hbm.at[pl.ds(base, _CH)])

    return body(y, pos0, pos1)


def kernel(inputs, gate_kernel, w0_kernel, w1_kernel, wo_kernel):
    inputs = inputs.astype(jnp.float32)
    x2 = inputs.reshape(-1, EMB)
    T = x2.shape[0]

    logits = jnp.einsum('bsd,de->bse', inputs, gate_kernel).reshape(T, NUM_EXPERTS)
    pos0, pos1, wa, wb, block_expert, vcnt, nb = _route(logits)
    nblocks = nb[0]

    xs, ws = _dispatch_sc(x2, pos0, pos1, wa, wb)

    inter = _up_gmm(block_expert, xs, w0_kernel, w1_kernel, nblocks)
    y = _dn_gmm(block_expert, vcnt, inter, wo_kernel, ws, nblocks)

    out = _combine_sc(y, pos0, pos1)
    return out.reshape(inputs.shape)
